# baseline (speedup 1.0000x reference)
import functools

import jax
import jax.numpy as jnp
from jax import lax
from jax.experimental import pallas as pl
from jax.experimental.pallas import tpu as pltpu
from jax.experimental.pallas import tpu_sc as plsc

N = 10000
E = 640000
G = 64
H = 128
NCLS = 5

NC = 2
NS = 16
EPC = E // NC
EPT = EPC // NS
ROWS_PT = 640
LAST_ROWS = N - 15 * ROWS_PT


def _make_sc_scatter(width, gather, ch, depth, ahead=0):
  n_full = ROWS_PT // ch
  n_last = LAST_ROWS // ch
  nchunk = EPT // ch
  ngrp = nchunk // depth

  def body(src_hbm, dst_hbm, tbl_hbm, out_hbm, sidx, didx, rows, acc, *sems):
    gsem = sems[:depth]
    ssem = sems[depth:]
    c = lax.axis_index("c")
    s = lax.axis_index("s")
    base = c * EPC + s * EPT
    row0 = s * ROWS_PT

    fill = jnp.zeros((16,), jnp.float32)

    @pl.loop(0, ch)
    def _zero_rows(i):
      for k in range(width // 16):
        rows[0, i, pl.ds(k * 16, 16)] = fill

    for r in range(n_full):
      @pl.when((s < NS - 1) | (r < n_last))
      def _zcp():
        pltpu.sync_copy(rows.at[0], acc.at[pl.ds(row0 + r * ch, ch)])

    if not gather:
      one = jnp.ones((16,), jnp.float32)

      @pl.loop(0, ch)
      def _fill_ones(i):
        for k in range(width // 16):
          rows[0, i, pl.ds(k * 16, 16)] = one

    plsc.subcore_barrier()

    if gather:
      def load_and_fire(chunk, k):
        off = pl.multiple_of(base + chunk * ch, 8)
        pltpu.sync_copy(dst_hbm.at[pl.ds(off, ch)], didx.at[k])
        pltpu.sync_copy(src_hbm.at[pl.ds(off, ch)], sidx.at[k])
        pltpu.async_copy(tbl_hbm.at[sidx.at[k]], rows.at[k], gsem[k])

      for k in range(ahead):
        load_and_fire(k, k)

      @pl.loop(0, ngrp)
      def _grp(g):
        for b in range(depth):
          cc = g * depth + b
          pltpu.make_async_copy(tbl_hbm.at[sidx.at[b]], rows.at[b],
                                gsem[b]).wait()
          pltpu.async_copy(rows.at[b], acc.at[didx.at[b]], ssem[b], add=True)
          kp = (b + ahead) % depth
          cp = cc + ahead

          @pl.when((cp < nchunk) & (cp >= depth))
          def _reuse():
            pltpu.make_async_copy(rows.at[kp], acc.at[didx.at[kp]],
                                  ssem[kp]).wait()

          @pl.when(cp < nchunk)
          def _prep():
            load_and_fire(cp, kp)

      for k in range(depth):
        pltpu.make_async_copy(rows.at[k], acc.at[didx.at[k]], ssem[k]).wait()
    else:
      def fire_idx(chunk, b):
        off = pl.multiple_of(base + chunk * ch, 8)
        pltpu.async_copy(dst_hbm.at[pl.ds(off, ch)], didx.at[b], sems[b])

      def wait_idx(chunk, b):
        off = pl.multiple_of(base + chunk * ch, 8)
        pltpu.make_async_copy(dst_hbm.at[pl.ds(off, ch)], didx.at[b],
                              sems[b]).wait()

      for b in range(depth):
        fire_idx(b, b)

      @pl.loop(0, ngrp)
      def _grp(g):
        for b in range(depth):
          chunk = g * depth + b
          wait_idx(chunk, b)
          pltpu.sync_copy(rows.at[0], acc.at[didx.at[b]], add=True)

          @pl.when(g + 1 < ngrp)
          def _prefetch():
            fire_idx(chunk + depth, b)

    plsc.subcore_barrier()

    @pl.when(s < NS - 1)
    def _wr_full():
      pltpu.sync_copy(acc.at[pl.ds(row0, ROWS_PT)],
                      out_hbm.at[c, pl.ds(row0, ROWS_PT)])

    @pl.when(s == NS - 1)
    def _wr_last():
      pltpu.sync_copy(acc.at[pl.ds(row0, LAST_ROWS)],
                      out_hbm.at[c, pl.ds(row0, LAST_ROWS)])

  mesh = plsc.VectorSubcoreMesh(core_axis_name="c", subcore_axis_name="s")
  return pl.kernel(
      body,
      compiler_params=pltpu.CompilerParams(use_tc_tiling_on_sc=False),
      out_type=jax.ShapeDtypeStruct((NC, N, width), jnp.float32),
      mesh=mesh,
      scratch_types=[
          pltpu.VMEM((depth, ch), jnp.int32),
          pltpu.VMEM((depth, ch), jnp.int32),
          pltpu.VMEM((depth, ch, width), jnp.float32),
          pltpu.VMEM_SHARED((N, width), jnp.float32),
      ] + [pltpu.SemaphoreType.DMA] * (2 * depth if gather else depth),
  )


_R = 2000
_NBLK = N // _R


def _tc_pre_body(degp_ref, xp_ref, dinv_ref, p1_ref):
  deg16 = degp_ref[0] + degp_ref[1] + 1.0
  dinv16 = lax.rsqrt(deg16)
  p1_ref[...] = dinv16 * xp_ref[...]
  dinv_ref[...] = dinv16[:, 0:1]


def _tc_layer1_body(sp_ref, p1_ref, dinv_ref, w1_ref, b1_ref, w2_ref, p2_ref):
  dinv = dinv_ref[...]
  z = dinv * (sp_ref[0] + sp_ref[1] + p1_ref[...])
  h = jnp.maximum(jnp.dot(z, w1_ref[...], preferred_element_type=jnp.float32)
                  + b1_ref[...], 0.0)
  p2_ref[...] = dinv * jnp.dot(h, w2_ref[...], preferred_element_type=jnp.float32)


def _tc_layer_body(sp_ref, p_ref, dinv_ref, b_ref, wn_ref, pn_ref):
  dinv = dinv_ref[...]
  h = jnp.maximum(dinv * (sp_ref[0] + sp_ref[1] + p_ref[...]) + b_ref[...], 0.0)
  pn_ref[...] = dinv * jnp.dot(h, wn_ref[...], preferred_element_type=jnp.float32)


def _tc_final_body(sp_ref, p_ref, dinv_ref, b_ref, batch_ref, wl_ref, bl_ref,
                   out_ref, sums, cnt):
  i = pl.program_id(0)

  @pl.when(i == 0)
  def _init():
    sums[...] = jnp.zeros_like(sums)
    cnt[...] = jnp.zeros_like(cnt)

  dinv = dinv_ref[...]
  h = jnp.maximum(dinv * (sp_ref[0] + sp_ref[1] + p_ref[...]) + b_ref[...], 0.0)
  iota = lax.broadcasted_iota(jnp.int32, (1, G), 1)
  oh = (batch_ref[...] == iota).astype(jnp.float32)
  dn = (((0,), (0,)), ((), ()))
  sums[...] += lax.dot_general(oh, h, dn, preferred_element_type=jnp.float32)
  cnt[...] += lax.dot_general(oh, jnp.ones((_R, H), jnp.float32), dn,
                              preferred_element_type=jnp.float32)

  @pl.when(i == _NBLK - 1)
  def _fin():
    pooled = sums[...] / jnp.maximum(cnt[...], 1.0)
    logits = jnp.dot(pooled, wl_ref[...], preferred_element_type=jnp.float32) + bl_ref[...]
    out_ref[...] = jax.nn.sigmoid(logits)


def _row_blocked(*shapes_maps):
  return [pl.BlockSpec(s, m) for s, m in shapes_maps]


def kernel(x, edge_index, batch, W1, b1, W2, b2, W3, b3, W4, b4, Wlin, blin):
  src = edge_index[0]
  dst = edge_index[1]

  xp = jnp.zeros((N, 16), jnp.float32).at[:, :4].set(x)
  W1p = jnp.zeros((16, H), jnp.float32).at[:4, :].set(W1)
  Wlp = jnp.zeros((H, 8), jnp.float32).at[:, :NCLS].set(Wlin)
  blp = jnp.zeros((1, 8), jnp.float32).at[0, :NCLS].set(blin)
  batch2 = batch.reshape(N, 1)

  deg_k = _make_sc_scatter(16, gather=False, ch=80, depth=5)
  spmm16 = _make_sc_scatter(16, gather=True, ch=80, depth=10, ahead=5)
  spmm128 = _make_sc_scatter(H, gather=True, ch=40, depth=5, ahead=2)

  degp = deg_k(src, dst, xp)

  pre = pl.pallas_call(
      _tc_pre_body,
      grid=(_NBLK,),
      in_specs=_row_blocked(((NC, _R, 16), lambda i: (0, i, 0)),
                            ((_R, 16), lambda i: (i, 0))),
      out_specs=_row_blocked(((_R, 1), lambda i: (i, 0)),
                             ((_R, 16), lambda i: (i, 0))),
      out_shape=[jax.ShapeDtypeStruct((N, 1), jnp.float32),
                 jax.ShapeDtypeStruct((N, 16), jnp.float32)],
  )
  dinv, p1 = pre(degp, xp)

  s1 = spmm16(src, dst, p1)

  l1 = pl.pallas_call(
      _tc_layer1_body,
      grid=(_NBLK,),
      in_specs=_row_blocked(((NC, _R, 16), lambda i: (0, i, 0)),
                            ((_R, 16), lambda i: (i, 0)),
                            ((_R, 1), lambda i: (i, 0)),
                            ((16, H), lambda i: (0, 0)),
                            ((1, H), lambda i: (0, 0)),
                            ((H, H), lambda i: (0, 0))),
      out_specs=pl.BlockSpec((_R, H), lambda i: (i, 0)),
      out_shape=jax.ShapeDtypeStruct((N, H), jnp.float32),
  )
  p2 = l1(s1, p1, dinv, W1p, b1.reshape(1, H), W2)

  layer = pl.pallas_call(
      _tc_layer_body,
      grid=(_NBLK,),
      in_specs=_row_blocked(((NC, _R, H), lambda i: (0, i, 0)),
                            ((_R, H), lambda i: (i, 0)),
                            ((_R, 1), lambda i: (i, 0)),
                            ((1, H), lambda i: (0, 0)),
                            ((H, H), lambda i: (0, 0))),
      out_specs=pl.BlockSpec((_R, H), lambda i: (i, 0)),
      out_shape=jax.ShapeDtypeStruct((N, H), jnp.float32),
  )

  s2 = spmm128(src, dst, p2)
  p3 = layer(s2, p2, dinv, b2.reshape(1, H), W3)
  s3 = spmm128(src, dst, p3)
  p4 = layer(s3, p3, dinv, b3.reshape(1, H), W4)
  s4 = spmm128(src, dst, p4)

  fin = pl.pallas_call(
      _tc_final_body,
      grid=(_NBLK,),
      in_specs=_row_blocked(((NC, _R, H), lambda i: (0, i, 0)),
                            ((_R, H), lambda i: (i, 0)),
                            ((_R, 1), lambda i: (i, 0)),
                            ((1, H), lambda i: (0, 0)),
                            ((_R, 1), lambda i: (i, 0)),
                            ((H, 8), lambda i: (0, 0)),
                            ((1, 8), lambda i: (0, 0))),
      out_specs=pl.BlockSpec((G, 8), lambda i: (0, 0)),
      out_shape=jax.ShapeDtypeStruct((G, 8), jnp.float32),
      scratch_shapes=[pltpu.VMEM((G, H), jnp.float32),
                      pltpu.VMEM((G, H), jnp.float32)],
  )
  out8 = fin(s4, p4, dinv, b4.reshape(1, H), batch2, Wlp, blp)
  return out8[:, :NCLS]


# speedup vs baseline: 1.4375x; 1.4375x over previous
import functools

import jax
import jax.numpy as jnp
from jax import lax
from jax.experimental import pallas as pl
from jax.experimental.pallas import tpu as pltpu
from jax.experimental.pallas import tpu_sc as plsc

N = 10000
E = 640000
G = 64
H = 128
NCLS = 5

NC = 2
NS = 16
EPC = E // NC
EPT = EPC // NS
ROWS_PT = 640
LAST_ROWS = N - 15 * ROWS_PT


def _make_sc_scatter(width, gather, ch, depth, ahead=0):
  n_full = ROWS_PT // ch
  n_last = LAST_ROWS // ch
  nchunk = EPT // ch
  rem = nchunk % depth
  ngrp = (nchunk - rem) // depth

  def body(src_hbm, dst_hbm, tbl_hbm, out_hbm, sidx, didx, rows, acc, *sems):
    gsem = sems[:depth]
    ssem = sems[depth:]
    c = lax.axis_index("c")
    s = lax.axis_index("s")
    base = c * EPC + s * EPT
    row0 = s * ROWS_PT

    fill = jnp.zeros((16,), jnp.float32)

    @pl.loop(0, ch)
    def _zero_rows(i):
      for k in range(width // 16):
        rows[0, i, pl.ds(k * 16, 16)] = fill

    for r in range(n_full):
      @pl.when((s < NS - 1) | (r < n_last))
      def _zcp():
        pltpu.sync_copy(rows.at[0], acc.at[pl.ds(row0 + r * ch, ch)])

    if not gather:
      one = jnp.ones((16,), jnp.float32)

      @pl.loop(0, ch)
      def _fill_ones(i):
        for k in range(width // 16):
          rows[0, i, pl.ds(k * 16, 16)] = one

    plsc.subcore_barrier()

    if gather:
      def load_and_fire(chunk, k):
        off = pl.multiple_of(base + chunk * ch, 8)
        pltpu.sync_copy(dst_hbm.at[pl.ds(off, ch)], didx.at[k])
        pltpu.sync_copy(src_hbm.at[pl.ds(off, ch)], sidx.at[k])
        pltpu.async_copy(tbl_hbm.at[sidx.at[k]], rows.at[k], gsem[k])

      for k in range(rem):
        load_and_fire(k, 0)
        pltpu.make_async_copy(tbl_hbm.at[sidx.at[0]], rows.at[0],
                              gsem[0]).wait()
        pltpu.sync_copy(rows.at[0], acc.at[didx.at[0]], add=True)

      for k in range(ahead):
        load_and_fire(rem + k, k)

      nmain = nchunk - rem

      @pl.loop(0, ngrp)
      def _grp(g):
        for b in range(depth):
          cc = g * depth + b
          pltpu.make_async_copy(tbl_hbm.at[sidx.at[b]], rows.at[b],
                                gsem[b]).wait()
          pltpu.async_copy(rows.at[b], acc.at[didx.at[b]], ssem[b], add=True)
          kp = (b + ahead) % depth
          cp = cc + ahead

          @pl.when((cp < nmain) & (cp >= depth))
          def _reuse():
            pltpu.make_async_copy(rows.at[kp], acc.at[didx.at[kp]],
                                  ssem[kp]).wait()

          @pl.when(cp < nmain)
          def _prep():
            load_and_fire(rem + cp, kp)

      for k in range(depth):
        pltpu.make_async_copy(rows.at[k], acc.at[didx.at[k]], ssem[k]).wait()
    else:
      def fire_idx(chunk, b):
        off = pl.multiple_of(base + chunk * ch, 8)
        pltpu.async_copy(dst_hbm.at[pl.ds(off, ch)], didx.at[b], sems[b])

      def wait_idx(chunk, b):
        off = pl.multiple_of(base + chunk * ch, 8)
        pltpu.make_async_copy(dst_hbm.at[pl.ds(off, ch)], didx.at[b],
                              sems[b]).wait()

      for b in range(depth):
        fire_idx(b, b)

      @pl.loop(0, ngrp)
      def _grp(g):
        for b in range(depth):
          chunk = g * depth + b
          wait_idx(chunk, b)
          pltpu.sync_copy(rows.at[0], acc.at[didx.at[b]], add=True)

          @pl.when(g + 1 < ngrp)
          def _prefetch():
            fire_idx(chunk + depth, b)

    plsc.subcore_barrier()

    @pl.when(s < NS - 1)
    def _wr_full():
      pltpu.sync_copy(acc.at[pl.ds(row0, ROWS_PT)],
                      out_hbm.at[c, pl.ds(row0, ROWS_PT)])

    @pl.when(s == NS - 1)
    def _wr_last():
      pltpu.sync_copy(acc.at[pl.ds(row0, LAST_ROWS)],
                      out_hbm.at[c, pl.ds(row0, LAST_ROWS)])

  mesh = plsc.VectorSubcoreMesh(core_axis_name="c", subcore_axis_name="s")
  return pl.kernel(
      body,
      compiler_params=pltpu.CompilerParams(use_tc_tiling_on_sc=False),
      out_type=jax.ShapeDtypeStruct((NC, N, width), jnp.float32),
      mesh=mesh,
      scratch_types=[
          pltpu.VMEM((depth, ch), jnp.int32),
          pltpu.VMEM((depth, ch), jnp.int32),
          pltpu.VMEM((depth, ch, width), jnp.float32),
          pltpu.VMEM_SHARED((N, width), jnp.float32),
      ] + [pltpu.SemaphoreType.DMA] * (2 * depth if gather else depth),
  )


_R = 2000
_NBLK = N // _R


def _tc_pre_body(degp_ref, xp_ref, dinv_ref, p1_ref):
  deg16 = degp_ref[0] + degp_ref[1] + 1.0
  dinv16 = lax.rsqrt(deg16)
  p1_ref[...] = dinv16 * xp_ref[...]
  dinv_ref[...] = dinv16[:, 0:1]


def _tc_layer1_body(sp_ref, p1_ref, dinv_ref, w1_ref, b1_ref, w2_ref, p2_ref):
  dinv = dinv_ref[...]
  z = dinv * (sp_ref[0] + sp_ref[1] + p1_ref[...])
  h = jnp.maximum(jnp.dot(z, w1_ref[...], preferred_element_type=jnp.float32)
                  + b1_ref[...], 0.0)
  p2_ref[...] = dinv * jnp.dot(h, w2_ref[...], preferred_element_type=jnp.float32)


def _tc_layer_body(sp_ref, p_ref, dinv_ref, b_ref, wn_ref, pn_ref):
  dinv = dinv_ref[...]
  h = jnp.maximum(dinv * (sp_ref[0] + sp_ref[1] + p_ref[...]) + b_ref[...], 0.0)
  pn_ref[...] = dinv * jnp.dot(h, wn_ref[...], preferred_element_type=jnp.float32)


def _tc_final_body(sp_ref, p_ref, dinv_ref, b_ref, batch_ref, wl_ref, bl_ref,
                   out_ref, sums, cnt):
  i = pl.program_id(0)

  @pl.when(i == 0)
  def _init():
    sums[...] = jnp.zeros_like(sums)
    cnt[...] = jnp.zeros_like(cnt)

  dinv = dinv_ref[...]
  h = jnp.maximum(dinv * (sp_ref[0] + sp_ref[1] + p_ref[...]) + b_ref[...], 0.0)
  iota = lax.broadcasted_iota(jnp.int32, (1, G), 1)
  oh = (batch_ref[...] == iota).astype(jnp.float32)
  dn = (((0,), (0,)), ((), ()))
  sums[...] += lax.dot_general(oh, h, dn, preferred_element_type=jnp.float32)
  cnt[...] += lax.dot_general(oh, jnp.ones((_R, H), jnp.float32), dn,
                              preferred_element_type=jnp.float32)

  @pl.when(i == _NBLK - 1)
  def _fin():
    pooled = sums[...] / jnp.maximum(cnt[...], 1.0)
    logits = jnp.dot(pooled, wl_ref[...], preferred_element_type=jnp.float32) + bl_ref[...]
    out_ref[...] = jax.nn.sigmoid(logits)


def _row_blocked(*shapes_maps):
  return [pl.BlockSpec(s, m) for s, m in shapes_maps]


def kernel(x, edge_index, batch, W1, b1, W2, b2, W3, b3, W4, b4, Wlin, blin):
  src = edge_index[0]
  dst = edge_index[1]

  xp = jnp.zeros((N, 16), jnp.float32).at[:, :4].set(x)
  W1p = jnp.zeros((16, H), jnp.float32).at[:4, :].set(W1)
  Wlp = jnp.zeros((H, 8), jnp.float32).at[:, :NCLS].set(Wlin)
  blp = jnp.zeros((1, 8), jnp.float32).at[0, :NCLS].set(blin)
  batch2 = batch.reshape(N, 1)

  deg_k = _make_sc_scatter(16, gather=False, ch=80, depth=5)
  spmm16 = _make_sc_scatter(16, gather=True, ch=80, depth=10, ahead=5)
  spmm128 = _make_sc_scatter(H, gather=True, ch=80, depth=4, ahead=2)

  degp = deg_k(src, dst, xp)

  pre = pl.pallas_call(
      _tc_pre_body,
      grid=(_NBLK,),
      in_specs=_row_blocked(((NC, _R, 16), lambda i: (0, i, 0)),
                            ((_R, 16), lambda i: (i, 0))),
      out_specs=_row_blocked(((_R, 1), lambda i: (i, 0)),
                             ((_R, 16), lambda i: (i, 0))),
      out_shape=[jax.ShapeDtypeStruct((N, 1), jnp.float32),
                 jax.ShapeDtypeStruct((N, 16), jnp.float32)],
  )
  dinv, p1 = pre(degp, xp)

  s1 = spmm16(src, dst, p1)

  l1 = pl.pallas_call(
      _tc_layer1_body,
      grid=(_NBLK,),
      in_specs=_row_blocked(((NC, _R, 16), lambda i: (0, i, 0)),
                            ((_R, 16), lambda i: (i, 0)),
                            ((_R, 1), lambda i: (i, 0)),
                            ((16, H), lambda i: (0, 0)),
                            ((1, H), lambda i: (0, 0)),
                            ((H, H), lambda i: (0, 0))),
      out_specs=pl.BlockSpec((_R, H), lambda i: (i, 0)),
      out_shape=jax.ShapeDtypeStruct((N, H), jnp.float32),
  )
  p2 = l1(s1, p1, dinv, W1p, b1.reshape(1, H), W2)

  layer = pl.pallas_call(
      _tc_layer_body,
      grid=(_NBLK,),
      in_specs=_row_blocked(((NC, _R, H), lambda i: (0, i, 0)),
                            ((_R, H), lambda i: (i, 0)),
                            ((_R, 1), lambda i: (i, 0)),
                            ((1, H), lambda i: (0, 0)),
                            ((H, H), lambda i: (0, 0))),
      out_specs=pl.BlockSpec((_R, H), lambda i: (i, 0)),
      out_shape=jax.ShapeDtypeStruct((N, H), jnp.float32),
  )

  s2 = spmm128(src, dst, p2)
  p3 = layer(s2, p2, dinv, b2.reshape(1, H), W3)
  s3 = spmm128(src, dst, p3)
  p4 = layer(s3, p3, dinv, b3.reshape(1, H), W4)
  s4 = spmm128(src, dst, p4)

  fin = pl.pallas_call(
      _tc_final_body,
      grid=(_NBLK,),
      in_specs=_row_blocked(((NC, _R, H), lambda i: (0, i, 0)),
                            ((_R, H), lambda i: (i, 0)),
                            ((_R, 1), lambda i: (i, 0)),
                            ((1, H), lambda i: (0, 0)),
                            ((_R, 1), lambda i: (i, 0)),
                            ((H, 8), lambda i: (0, 0)),
                            ((1, 8), lambda i: (0, 0))),
      out_specs=pl.BlockSpec((G, 8), lambda i: (0, 0)),
      out_shape=jax.ShapeDtypeStruct((G, 8), jnp.float32),
      scratch_shapes=[pltpu.VMEM((G, H), jnp.float32),
                      pltpu.VMEM((G, H), jnp.float32)],
  )
  out8 = fin(s4, p4, dinv, b4.reshape(1, H), batch2, Wlp, blp)
  return out8[:, :NCLS]


# speedup vs baseline: 1.7879x; 1.2437x over previous
import functools

import jax
import jax.numpy as jnp
from jax import lax
from jax.experimental import pallas as pl
from jax.experimental.pallas import tpu as pltpu
from jax.experimental.pallas import tpu_sc as plsc

N = 10000
E = 640000
G = 64
H = 128
NCLS = 5

NC = 2
NS = 16
EPC = E // NC
EPT = EPC // NS
ROWS_PT = 640
LAST_ROWS = N - 15 * ROWS_PT


def _make_sc_scatter(width, gather, ch, depth, ahead=0):
  n_full = ROWS_PT // ch
  n_last = LAST_ROWS // ch
  nchunk = EPT // ch
  rem = nchunk % depth
  ngrp = (nchunk - rem) // depth

  def body(eidx_hbm, tbl_hbm, out_hbm, idx2, rows, acc, *sems):
    gsem = sems[:depth]
    ssem = sems[depth:]
    c = lax.axis_index("c")
    s = lax.axis_index("s")
    cbase = c * (EPC // ch) + s * nchunk
    row0 = s * ROWS_PT

    fill = jnp.zeros((16,), jnp.float32)

    @pl.loop(0, ch)
    def _zero_rows(i):
      for k in range(width // 16):
        rows[0, i, pl.ds(k * 16, 16)] = fill

    for r in range(n_full):
      @pl.when((s < NS - 1) | (r < n_last))
      def _zcp():
        pltpu.sync_copy(rows.at[0], acc.at[pl.ds(row0 + r * ch, ch)])

    if not gather:
      one = jnp.ones((16,), jnp.float32)

      @pl.loop(0, ch)
      def _fill_ones(i):
        for k in range(width // 16):
          rows[0, i, pl.ds(k * 16, 16)] = one

    plsc.subcore_barrier()

    if gather:
      def load_and_fire(chunk, k):
        pltpu.sync_copy(eidx_hbm.at[cbase + chunk], idx2.at[k])
        pltpu.async_copy(tbl_hbm.at[idx2.at[k, 0]], rows.at[k], gsem[k])

      for k in range(rem):
        load_and_fire(k, 0)
        pltpu.make_async_copy(tbl_hbm.at[idx2.at[0, 0]], rows.at[0],
                              gsem[0]).wait()
        pltpu.sync_copy(rows.at[0], acc.at[idx2.at[0, 1]], add=True)

      for k in range(ahead):
        load_and_fire(rem + k, k)

      nmain = nchunk - rem

      @pl.loop(0, ngrp)
      def _grp(g):
        for b in range(depth):
          cc = g * depth + b
          pltpu.make_async_copy(tbl_hbm.at[idx2.at[b, 0]], rows.at[b],
                                gsem[b]).wait()
          pltpu.async_copy(rows.at[b], acc.at[idx2.at[b, 1]], ssem[b],
                           add=True)
          kp = (b + ahead) % depth
          cp = cc + ahead

          @pl.when((cp < nmain) & (cp >= depth))
          def _reuse():
            pltpu.make_async_copy(rows.at[kp], acc.at[idx2.at[kp, 1]],
                                  ssem[kp]).wait()

          @pl.when(cp < nmain)
          def _prep():
            load_and_fire(rem + cp, kp)

      for k in range(depth):
        pltpu.make_async_copy(rows.at[k], acc.at[idx2.at[k, 1]],
                              ssem[k]).wait()
    else:
      def fire_idx(chunk, b):
        pltpu.async_copy(eidx_hbm.at[cbase + chunk], idx2.at[b], sems[b])

      def wait_idx(chunk, b):
        pltpu.make_async_copy(eidx_hbm.at[cbase + chunk], idx2.at[b],
                              sems[b]).wait()

      for b in range(depth):
        fire_idx(b, b)

      @pl.loop(0, ngrp)
      def _grp(g):
        for b in range(depth):
          chunk = g * depth + b
          wait_idx(chunk, b)
          pltpu.sync_copy(rows.at[0], acc.at[idx2.at[b, 1]], add=True)

          @pl.when(g + 1 < ngrp)
          def _prefetch():
            fire_idx(chunk + depth, b)

    plsc.subcore_barrier()

    @pl.when(s < NS - 1)
    def _wr_full():
      pltpu.sync_copy(acc.at[pl.ds(row0, ROWS_PT)],
                      out_hbm.at[c, pl.ds(row0, ROWS_PT)])

    @pl.when(s == NS - 1)
    def _wr_last():
      pltpu.sync_copy(acc.at[pl.ds(row0, LAST_ROWS)],
                      out_hbm.at[c, pl.ds(row0, LAST_ROWS)])

  mesh = plsc.VectorSubcoreMesh(core_axis_name="c", subcore_axis_name="s")
  return pl.kernel(
      body,
      compiler_params=pltpu.CompilerParams(use_tc_tiling_on_sc=False),
      out_type=jax.ShapeDtypeStruct((NC, N, width), jnp.float32),
      mesh=mesh,
      scratch_types=[
          pltpu.VMEM((depth, 2, ch), jnp.int32),
          pltpu.VMEM((depth, ch, width), jnp.float32),
          pltpu.VMEM_SHARED((N, width), jnp.float32),
      ] + [pltpu.SemaphoreType.DMA] * (2 * depth if gather else depth),
  )


_R = 2000
_NBLK = N // _R


def _tc_pre_body(degp_ref, xp_ref, dinv_ref, p1_ref):
  deg16 = degp_ref[0] + degp_ref[1] + 1.0
  dinv16 = lax.rsqrt(deg16)
  p1_ref[...] = dinv16 * xp_ref[...]
  dinv_ref[...] = dinv16[:, 0:1]


def _tc_layer1_body(sp_ref, p1_ref, dinv_ref, w1_ref, b1_ref, w2_ref, p2_ref):
  dinv = dinv_ref[...]
  z = dinv * (sp_ref[0] + sp_ref[1] + p1_ref[...])
  h = jnp.maximum(jnp.dot(z, w1_ref[...], preferred_element_type=jnp.float32)
                  + b1_ref[...], 0.0)
  p2_ref[...] = dinv * jnp.dot(h, w2_ref[...], preferred_element_type=jnp.float32)


def _tc_layer_body(sp_ref, p_ref, dinv_ref, b_ref, wn_ref, pn_ref):
  dinv = dinv_ref[...]
  h = jnp.maximum(dinv * (sp_ref[0] + sp_ref[1] + p_ref[...]) + b_ref[...], 0.0)
  pn_ref[...] = dinv * jnp.dot(h, wn_ref[...], preferred_element_type=jnp.float32)


def _tc_final_body(sp_ref, p_ref, dinv_ref, b_ref, batch_ref, wl_ref, bl_ref,
                   out_ref, sums, cnt):
  i = pl.program_id(0)

  @pl.when(i == 0)
  def _init():
    sums[...] = jnp.zeros_like(sums)
    cnt[...] = jnp.zeros_like(cnt)

  dinv = dinv_ref[...]
  h = jnp.maximum(dinv * (sp_ref[0] + sp_ref[1] + p_ref[...]) + b_ref[...], 0.0)
  iota = lax.broadcasted_iota(jnp.int32, (1, G), 1)
  oh = (batch_ref[...] == iota).astype(jnp.float32)
  dn = (((0,), (0,)), ((), ()))
  sums[...] += lax.dot_general(oh, h, dn, preferred_element_type=jnp.float32)
  cnt[...] += lax.dot_general(oh, jnp.ones((_R, H), jnp.float32), dn,
                              preferred_element_type=jnp.float32)

  @pl.when(i == _NBLK - 1)
  def _fin():
    pooled = sums[...] / jnp.maximum(cnt[...], 1.0)
    logits = jnp.dot(pooled, wl_ref[...], preferred_element_type=jnp.float32) + bl_ref[...]
    out_ref[...] = jax.nn.sigmoid(logits)


def _row_blocked(*shapes_maps):
  return [pl.BlockSpec(s, m) for s, m in shapes_maps]


def kernel(x, edge_index, batch, W1, b1, W2, b2, W3, b3, W4, b4, Wlin, blin):
  src = edge_index[0]
  dst = edge_index[1]
  pk80 = jnp.stack([src.reshape(-1, 80), dst.reshape(-1, 80)], axis=1)
  pk200 = jnp.stack([src.reshape(-1, 200), dst.reshape(-1, 200)], axis=1)

  xp = jnp.zeros((N, 16), jnp.float32).at[:, :4].set(x)
  W1p = jnp.zeros((16, H), jnp.float32).at[:4, :].set(W1)
  Wlp = jnp.zeros((H, 8), jnp.float32).at[:, :NCLS].set(Wlin)
  blp = jnp.zeros((1, 8), jnp.float32).at[0, :NCLS].set(blin)
  batch2 = batch.reshape(N, 1)

  deg_k = _make_sc_scatter(16, gather=False, ch=200, depth=5)
  spmm16 = _make_sc_scatter(16, gather=True, ch=200, depth=10, ahead=5)
  spmm128 = _make_sc_scatter(H, gather=True, ch=80, depth=4, ahead=2)

  degp = deg_k(pk200, xp)

  pre = pl.pallas_call(
      _tc_pre_body,
      grid=(_NBLK,),
      in_specs=_row_blocked(((NC, _R, 16), lambda i: (0, i, 0)),
                            ((_R, 16), lambda i: (i, 0))),
      out_specs=_row_blocked(((_R, 1), lambda i: (i, 0)),
                             ((_R, 16), lambda i: (i, 0))),
      out_shape=[jax.ShapeDtypeStruct((N, 1), jnp.float32),
                 jax.ShapeDtypeStruct((N, 16), jnp.float32)],
  )
  dinv, p1 = pre(degp, xp)

  s1 = spmm16(pk200, p1)

  l1 = pl.pallas_call(
      _tc_layer1_body,
      grid=(_NBLK,),
      in_specs=_row_blocked(((NC, _R, 16), lambda i: (0, i, 0)),
                            ((_R, 16), lambda i: (i, 0)),
                            ((_R, 1), lambda i: (i, 0)),
                            ((16, H), lambda i: (0, 0)),
                            ((1, H), lambda i: (0, 0)),
                            ((H, H), lambda i: (0, 0))),
      out_specs=pl.BlockSpec((_R, H), lambda i: (i, 0)),
      out_shape=jax.ShapeDtypeStruct((N, H), jnp.float32),
  )
  p2 = l1(s1, p1, dinv, W1p, b1.reshape(1, H), W2)

  layer = pl.pallas_call(
      _tc_layer_body,
      grid=(_NBLK,),
      in_specs=_row_blocked(((NC, _R, H), lambda i: (0, i, 0)),
                            ((_R, H), lambda i: (i, 0)),
                            ((_R, 1), lambda i: (i, 0)),
                            ((1, H), lambda i: (0, 0)),
                            ((H, H), lambda i: (0, 0))),
      out_specs=pl.BlockSpec((_R, H), lambda i: (i, 0)),
      out_shape=jax.ShapeDtypeStruct((N, H), jnp.float32),
  )

  s2 = spmm128(pk80, p2)
  p3 = layer(s2, p2, dinv, b2.reshape(1, H), W3)
  s3 = spmm128(pk80, p3)
  p4 = layer(s3, p3, dinv, b3.reshape(1, H), W4)
  s4 = spmm128(pk80, p4)

  fin = pl.pallas_call(
      _tc_final_body,
      grid=(_NBLK,),
      in_specs=_row_blocked(((NC, _R, H), lambda i: (0, i, 0)),
                            ((_R, H), lambda i: (i, 0)),
                            ((_R, 1), lambda i: (i, 0)),
                            ((1, H), lambda i: (0, 0)),
                            ((_R, 1), lambda i: (i, 0)),
                            ((H, 8), lambda i: (0, 0)),
                            ((1, 8), lambda i: (0, 0))),
      out_specs=pl.BlockSpec((G, 8), lambda i: (0, 0)),
      out_shape=jax.ShapeDtypeStruct((G, 8), jnp.float32),
      scratch_shapes=[pltpu.VMEM((G, H), jnp.float32),
                      pltpu.VMEM((G, H), jnp.float32)],
  )
  out8 = fin(s4, p4, dinv, b4.reshape(1, H), batch2, Wlp, blp)
  return out8[:, :NCLS]


# speedup vs baseline: 1.7901x; 1.0013x over previous
import functools

import jax
import jax.numpy as jnp
from jax import lax
from jax.experimental import pallas as pl
from jax.experimental.pallas import tpu as pltpu
from jax.experimental.pallas import tpu_sc as plsc

N = 10000
E = 640000
G = 64
H = 128
NCLS = 5

NC = 2
NS = 16
EPC = E // NC
EPT = EPC // NS
ROWS_PT = 640
LAST_ROWS = N - 15 * ROWS_PT


def _make_sc_scatter(width, gather, ch, depth, ahead=0):
  ZCH = 80
  n_full = ROWS_PT // ZCH
  n_last = LAST_ROWS // ZCH
  nchunk = EPT // ch
  rem = nchunk % depth
  ngrp = (nchunk - rem) // depth

  def body(eidx_hbm, tbl_hbm, out_hbm, idx2, rows, acc, *sems):
    gsem = sems[:depth]
    ssem = sems[depth:]
    c = lax.axis_index("c")
    s = lax.axis_index("s")
    cbase = c * (EPC // ch) + s * nchunk
    row0 = s * ROWS_PT

    fill = jnp.zeros((16,), jnp.float32)

    @pl.loop(0, ZCH)
    def _zero_rows(i):
      for k in range(width // 16):
        rows[0, i, pl.ds(k * 16, 16)] = fill

    for r in range(n_full):
      @pl.when((s < NS - 1) | (r < n_last))
      def _zcp():
        pltpu.sync_copy(rows.at[0, pl.ds(0, ZCH)],
                        acc.at[pl.ds(row0 + r * ZCH, ZCH)])

    if not gather:
      one = jnp.ones((16,), jnp.float32)

      @pl.loop(0, ch)
      def _fill_ones(i):
        for k in range(width // 16):
          rows[0, i, pl.ds(k * 16, 16)] = one

    plsc.subcore_barrier()

    if gather:
      def load_and_fire(chunk, k):
        pltpu.sync_copy(eidx_hbm.at[cbase + chunk], idx2.at[k])
        pltpu.async_copy(tbl_hbm.at[idx2.at[k, 0]], rows.at[k], gsem[k])

      for k in range(rem):
        load_and_fire(k, 0)
        pltpu.make_async_copy(tbl_hbm.at[idx2.at[0, 0]], rows.at[0],
                              gsem[0]).wait()
        pltpu.sync_copy(rows.at[0], acc.at[idx2.at[0, 1]], add=True)

      for k in range(ahead):
        load_and_fire(rem + k, k)

      nmain = nchunk - rem

      @pl.loop(0, ngrp)
      def _grp(g):
        for b in range(depth):
          cc = g * depth + b
          pltpu.make_async_copy(tbl_hbm.at[idx2.at[b, 0]], rows.at[b],
                                gsem[b]).wait()
          pltpu.async_copy(rows.at[b], acc.at[idx2.at[b, 1]], ssem[b],
                           add=True)
          kp = (b + ahead) % depth
          cp = cc + ahead

          @pl.when((cp < nmain) & (cp >= depth))
          def _reuse():
            pltpu.make_async_copy(rows.at[kp], acc.at[idx2.at[kp, 1]],
                                  ssem[kp]).wait()

          @pl.when(cp < nmain)
          def _prep():
            load_and_fire(rem + cp, kp)

      for k in range(depth):
        pltpu.make_async_copy(rows.at[k], acc.at[idx2.at[k, 1]],
                              ssem[k]).wait()
    else:
      def fire_idx(chunk, b):
        pltpu.async_copy(eidx_hbm.at[cbase + chunk], idx2.at[b], sems[b])

      def wait_idx(chunk, b):
        pltpu.make_async_copy(eidx_hbm.at[cbase + chunk], idx2.at[b],
                              sems[b]).wait()

      for b in range(depth):
        fire_idx(b, b)

      @pl.loop(0, ngrp)
      def _grp(g):
        for b in range(depth):
          chunk = g * depth + b
          wait_idx(chunk, b)
          pltpu.sync_copy(rows.at[0], acc.at[idx2.at[b, 1]], add=True)

          @pl.when(g + 1 < ngrp)
          def _prefetch():
            fire_idx(chunk + depth, b)

    plsc.subcore_barrier()

    @pl.when(s < NS - 1)
    def _wr_full():
      pltpu.sync_copy(acc.at[pl.ds(row0, ROWS_PT)],
                      out_hbm.at[c, pl.ds(row0, ROWS_PT)])

    @pl.when(s == NS - 1)
    def _wr_last():
      pltpu.sync_copy(acc.at[pl.ds(row0, LAST_ROWS)],
                      out_hbm.at[c, pl.ds(row0, LAST_ROWS)])

  mesh = plsc.VectorSubcoreMesh(core_axis_name="c", subcore_axis_name="s")
  return pl.kernel(
      body,
      compiler_params=pltpu.CompilerParams(use_tc_tiling_on_sc=False),
      out_type=jax.ShapeDtypeStruct((NC, N, width), jnp.float32),
      mesh=mesh,
      scratch_types=[
          pltpu.VMEM((depth, 2, ch), jnp.int32),
          pltpu.VMEM((depth, ch, width), jnp.float32),
          pltpu.VMEM_SHARED((N, width), jnp.float32),
      ] + [pltpu.SemaphoreType.DMA] * (2 * depth if gather else depth),
  )


_R = 2000
_NBLK = N // _R


def _tc_pre_body(degp_ref, xp_ref, dinv_ref, p1_ref):
  deg16 = degp_ref[0] + degp_ref[1] + 1.0
  dinv16 = lax.rsqrt(deg16)
  p1_ref[...] = dinv16 * xp_ref[...]
  dinv_ref[...] = dinv16[:, 0:1]


def _tc_layer1_body(sp_ref, p1_ref, dinv_ref, w1_ref, b1_ref, w2_ref, p2_ref):
  dinv = dinv_ref[...]
  z = dinv * (sp_ref[0] + sp_ref[1] + p1_ref[...])
  h = jnp.maximum(jnp.dot(z, w1_ref[...], preferred_element_type=jnp.float32)
                  + b1_ref[...], 0.0)
  p2_ref[...] = dinv * jnp.dot(h, w2_ref[...], preferred_element_type=jnp.float32)


def _tc_layer_body(sp_ref, p_ref, dinv_ref, b_ref, wn_ref, pn_ref):
  dinv = dinv_ref[...]
  h = jnp.maximum(dinv * (sp_ref[0] + sp_ref[1] + p_ref[...]) + b_ref[...], 0.0)
  pn_ref[...] = dinv * jnp.dot(h, wn_ref[...], preferred_element_type=jnp.float32)


def _tc_final_body(sp_ref, p_ref, dinv_ref, b_ref, batch_ref, wl_ref, bl_ref,
                   out_ref, sums, cnt):
  i = pl.program_id(0)

  @pl.when(i == 0)
  def _init():
    sums[...] = jnp.zeros_like(sums)
    cnt[...] = jnp.zeros_like(cnt)

  dinv = dinv_ref[...]
  h = jnp.maximum(dinv * (sp_ref[0] + sp_ref[1] + p_ref[...]) + b_ref[...], 0.0)
  iota = lax.broadcasted_iota(jnp.int32, (1, G), 1)
  oh = (batch_ref[...] == iota).astype(jnp.float32)
  dn = (((0,), (0,)), ((), ()))
  sums[...] += lax.dot_general(oh, h, dn, preferred_element_type=jnp.float32)
  cnt[...] += lax.dot_general(oh, jnp.ones((_R, H), jnp.float32), dn,
                              preferred_element_type=jnp.float32)

  @pl.when(i == _NBLK - 1)
  def _fin():
    pooled = sums[...] / jnp.maximum(cnt[...], 1.0)
    logits = jnp.dot(pooled, wl_ref[...], preferred_element_type=jnp.float32) + bl_ref[...]
    out_ref[...] = jax.nn.sigmoid(logits)


def _row_blocked(*shapes_maps):
  return [pl.BlockSpec(s, m) for s, m in shapes_maps]


def kernel(x, edge_index, batch, W1, b1, W2, b2, W3, b3, W4, b4, Wlin, blin):
  src = edge_index[0]
  dst = edge_index[1]
  pk80 = jnp.stack([src.reshape(-1, 80), dst.reshape(-1, 80)], axis=1)
  pk200 = jnp.stack([src.reshape(-1, 200), dst.reshape(-1, 200)], axis=1)

  xp = jnp.zeros((N, 16), jnp.float32).at[:, :4].set(x)
  W1p = jnp.zeros((16, H), jnp.float32).at[:4, :].set(W1)
  Wlp = jnp.zeros((H, 8), jnp.float32).at[:, :NCLS].set(Wlin)
  blp = jnp.zeros((1, 8), jnp.float32).at[0, :NCLS].set(blin)
  batch2 = batch.reshape(N, 1)

  deg_k = _make_sc_scatter(16, gather=False, ch=200, depth=5)
  spmm16 = _make_sc_scatter(16, gather=True, ch=200, depth=10, ahead=5)
  spmm128 = _make_sc_scatter(H, gather=True, ch=80, depth=4, ahead=2)

  degp = deg_k(pk200, xp)

  pre = pl.pallas_call(
      _tc_pre_body,
      grid=(_NBLK,),
      in_specs=_row_blocked(((NC, _R, 16), lambda i: (0, i, 0)),
                            ((_R, 16), lambda i: (i, 0))),
      out_specs=_row_blocked(((_R, 1), lambda i: (i, 0)),
                             ((_R, 16), lambda i: (i, 0))),
      out_shape=[jax.ShapeDtypeStruct((N, 1), jnp.float32),
                 jax.ShapeDtypeStruct((N, 16), jnp.float32)],
  )
  dinv, p1 = pre(degp, xp)

  s1 = spmm16(pk200, p1)

  l1 = pl.pallas_call(
      _tc_layer1_body,
      grid=(_NBLK,),
      in_specs=_row_blocked(((NC, _R, 16), lambda i: (0, i, 0)),
                            ((_R, 16), lambda i: (i, 0)),
                            ((_R, 1), lambda i: (i, 0)),
                            ((16, H), lambda i: (0, 0)),
                            ((1, H), lambda i: (0, 0)),
                            ((H, H), lambda i: (0, 0))),
      out_specs=pl.BlockSpec((_R, H), lambda i: (i, 0)),
      out_shape=jax.ShapeDtypeStruct((N, H), jnp.float32),
  )
  p2 = l1(s1, p1, dinv, W1p, b1.reshape(1, H), W2)

  layer = pl.pallas_call(
      _tc_layer_body,
      grid=(_NBLK,),
      in_specs=_row_blocked(((NC, _R, H), lambda i: (0, i, 0)),
                            ((_R, H), lambda i: (i, 0)),
                            ((_R, 1), lambda i: (i, 0)),
                            ((1, H), lambda i: (0, 0)),
                            ((H, H), lambda i: (0, 0))),
      out_specs=pl.BlockSpec((_R, H), lambda i: (i, 0)),
      out_shape=jax.ShapeDtypeStruct((N, H), jnp.float32),
  )

  s2 = spmm128(pk80, p2)
  p3 = layer(s2, p2, dinv, b2.reshape(1, H), W3)
  s3 = spmm128(pk80, p3)
  p4 = layer(s3, p3, dinv, b3.reshape(1, H), W4)
  s4 = spmm128(pk80, p4)

  fin = pl.pallas_call(
      _tc_final_body,
      grid=(_NBLK,),
      in_specs=_row_blocked(((NC, _R, H), lambda i: (0, i, 0)),
                            ((_R, H), lambda i: (i, 0)),
                            ((_R, 1), lambda i: (i, 0)),
                            ((1, H), lambda i: (0, 0)),
                            ((_R, 1), lambda i: (i, 0)),
                            ((H, 8), lambda i: (0, 0)),
                            ((1, 8), lambda i: (0, 0))),
      out_specs=pl.BlockSpec((G, 8), lambda i: (0, 0)),
      out_shape=jax.ShapeDtypeStruct((G, 8), jnp.float32),
      scratch_shapes=[pltpu.VMEM((G, H), jnp.float32),
                      pltpu.VMEM((G, H), jnp.float32)],
  )
  out8 = fin(s4, p4, dinv, b4.reshape(1, H), batch2, Wlp, blp)
  return out8[:, :NCLS]


# speedup vs baseline: 1.9010x; 1.0619x over previous
import functools

import jax
import jax.numpy as jnp
from jax import lax
from jax.experimental import pallas as pl
from jax.experimental.pallas import tpu as pltpu
from jax.experimental.pallas import tpu_sc as plsc

N = 10000
E = 640000
G = 64
H = 128
NCLS = 5

NC = 2
NS = 16
EPC = E // NC
EPT = EPC // NS
ROWS_PT = 640
LAST_ROWS = N - 15 * ROWS_PT


def _make_sc_scatter(width, gather, ch, depth, ahead=0):
  ZCH = 80
  n_full = ROWS_PT // ZCH
  n_last = LAST_ROWS // ZCH
  nchunk = EPT // ch
  rem = nchunk % depth
  ngrp = (nchunk - rem) // depth

  def body(eidx_hbm, tbl_hbm, out_hbm, idx2, rows, acc, *sems):
    gsem = sems[:depth]
    ssem = sems[depth:]
    c = lax.axis_index("c")
    s = lax.axis_index("s")
    cbase = c * (EPC // ch) + s * nchunk
    row0 = s * ROWS_PT

    fill = jnp.zeros((16,), jnp.float32)

    @pl.loop(0, ZCH)
    def _zero_rows(i):
      for k in range(width // 16):
        rows[0, i, pl.ds(k * 16, 16)] = fill

    for r in range(n_full):
      @pl.when((s < NS - 1) | (r < n_last))
      def _zcp():
        pltpu.sync_copy(rows.at[0, pl.ds(0, ZCH)],
                        acc.at[pl.ds(row0 + r * ZCH, ZCH)])

    if not gather:
      one = jnp.ones((16,), jnp.float32)

      @pl.loop(0, ch)
      def _fill_ones(i):
        for k in range(width // 16):
          rows[0, i, pl.ds(k * 16, 16)] = one

    plsc.subcore_barrier()

    if gather:
      def load_and_fire(chunk, k):
        pltpu.sync_copy(eidx_hbm.at[cbase + chunk], idx2.at[k])
        pltpu.async_copy(tbl_hbm.at[idx2.at[k, 0]], rows.at[k], gsem[k])

      for k in range(rem):
        load_and_fire(k, 0)
        pltpu.make_async_copy(tbl_hbm.at[idx2.at[0, 0]], rows.at[0],
                              gsem[0]).wait()
        pltpu.sync_copy(rows.at[0], acc.at[idx2.at[0, 1]], add=True)

      for k in range(ahead):
        load_and_fire(rem + k, k)

      nmain = nchunk - rem

      @pl.loop(0, ngrp)
      def _grp(g):
        for b in range(depth):
          cc = g * depth + b
          pltpu.make_async_copy(tbl_hbm.at[idx2.at[b, 0]], rows.at[b],
                                gsem[b]).wait()
          pltpu.async_copy(rows.at[b], acc.at[idx2.at[b, 1]], ssem[b],
                           add=True)
          kp = (b + ahead) % depth
          cp = cc + ahead

          @pl.when((cp < nmain) & (cp >= depth))
          def _reuse():
            pltpu.make_async_copy(rows.at[kp], acc.at[idx2.at[kp, 1]],
                                  ssem[kp]).wait()

          @pl.when(cp < nmain)
          def _prep():
            load_and_fire(rem + cp, kp)

      for k in range(depth):
        pltpu.make_async_copy(rows.at[k], acc.at[idx2.at[k, 1]],
                              ssem[k]).wait()
    else:
      def fire_idx(chunk, b):
        pltpu.async_copy(eidx_hbm.at[cbase + chunk], idx2.at[b], sems[b])

      def wait_idx(chunk, b):
        pltpu.make_async_copy(eidx_hbm.at[cbase + chunk], idx2.at[b],
                              sems[b]).wait()

      for b in range(depth):
        fire_idx(b, b)

      @pl.loop(0, ngrp)
      def _grp(g):
        for b in range(depth):
          chunk = g * depth + b
          wait_idx(chunk, b)
          pltpu.sync_copy(rows.at[0], acc.at[idx2.at[b, 1]], add=True)

          @pl.when(g + 1 < ngrp)
          def _prefetch():
            fire_idx(chunk + depth, b)

    plsc.subcore_barrier()

    @pl.when(s < NS - 1)
    def _wr_full():
      pltpu.sync_copy(acc.at[pl.ds(row0, ROWS_PT)],
                      out_hbm.at[c, pl.ds(row0, ROWS_PT)])

    @pl.when(s == NS - 1)
    def _wr_last():
      pltpu.sync_copy(acc.at[pl.ds(row0, LAST_ROWS)],
                      out_hbm.at[c, pl.ds(row0, LAST_ROWS)])

  mesh = plsc.VectorSubcoreMesh(core_axis_name="c", subcore_axis_name="s")
  return pl.kernel(
      body,
      compiler_params=pltpu.CompilerParams(use_tc_tiling_on_sc=False),
      out_type=jax.ShapeDtypeStruct((NC, N, width), jnp.float32),
      mesh=mesh,
      scratch_types=[
          pltpu.VMEM((depth, 2, ch), jnp.int32),
          pltpu.VMEM((depth, ch, width), jnp.float32),
          pltpu.VMEM_SHARED((N, width), jnp.float32),
      ] + [pltpu.SemaphoreType.DMA] * (2 * depth if gather else depth),
  )


_R = 2000
_NBLK = N // _R


def _tc_pre_body(degp_ref, xp_ref, dinv_ref, p1_ref):
  deg16 = degp_ref[0] + degp_ref[1] + 1.0
  dinv16 = lax.rsqrt(deg16)
  p1_ref[...] = dinv16 * xp_ref[...]
  dinv_ref[...] = dinv16[:, 0:1]


def _tc_layer1_body(sp_ref, p1_ref, dinv_ref, w1_ref, b1_ref, w2_ref, p2_ref):
  dinv = dinv_ref[...]
  z = dinv * (sp_ref[0] + sp_ref[1] + p1_ref[...])
  h = jnp.maximum(jnp.dot(z, w1_ref[...], preferred_element_type=jnp.float32)
                  + b1_ref[...], 0.0)
  p2_ref[...] = dinv * jnp.dot(h, w2_ref[...], preferred_element_type=jnp.float32)


def _tc_layer_body(sp_ref, p_ref, dinv_ref, b_ref, wn_ref, pn_ref):
  dinv = dinv_ref[...]
  h = jnp.maximum(dinv * (sp_ref[0] + sp_ref[1] + p_ref[...]) + b_ref[...], 0.0)
  pn_ref[...] = dinv * jnp.dot(h, wn_ref[...], preferred_element_type=jnp.float32)


def _tc_final_body(sp_ref, p_ref, dinv_ref, b_ref, batch_ref, wl_ref, bl_ref,
                   out_ref, sums, cnt):
  i = pl.program_id(0)

  @pl.when(i == 0)
  def _init():
    sums[...] = jnp.zeros_like(sums)
    cnt[...] = jnp.zeros_like(cnt)

  dinv = dinv_ref[...]
  h = jnp.maximum(dinv * (sp_ref[0] + sp_ref[1] + p_ref[...]) + b_ref[...], 0.0)
  iota = lax.broadcasted_iota(jnp.int32, (1, G), 1)
  oh = (batch_ref[...] == iota).astype(jnp.float32)
  dn = (((0,), (0,)), ((), ()))
  sums[...] += lax.dot_general(oh, h, dn, preferred_element_type=jnp.float32)
  cnt[...] += lax.dot_general(oh, jnp.ones((_R, H), jnp.float32), dn,
                              preferred_element_type=jnp.float32)

  @pl.when(i == _NBLK - 1)
  def _fin():
    pooled = sums[...] / jnp.maximum(cnt[...], 1.0)
    logits = jnp.dot(pooled, wl_ref[...], preferred_element_type=jnp.float32) + bl_ref[...]
    out_ref[...] = jax.nn.sigmoid(logits)


def _row_blocked(*shapes_maps):
  return [pl.BlockSpec(s, m) for s, m in shapes_maps]


def kernel(x, edge_index, batch, W1, b1, W2, b2, W3, b3, W4, b4, Wlin, blin):
  src = edge_index[0]
  dst = edge_index[1]
  pk160 = jnp.stack([src.reshape(-1, 160), dst.reshape(-1, 160)], axis=1)
  pk200 = jnp.stack([src.reshape(-1, 200), dst.reshape(-1, 200)], axis=1)

  xp = jnp.zeros((N, 16), jnp.float32).at[:, :4].set(x)
  W1p = jnp.zeros((16, H), jnp.float32).at[:4, :].set(W1)
  Wlp = jnp.zeros((H, 8), jnp.float32).at[:, :NCLS].set(Wlin)
  blp = jnp.zeros((1, 8), jnp.float32).at[0, :NCLS].set(blin)
  batch2 = batch.reshape(N, 1)

  deg_k = _make_sc_scatter(16, gather=False, ch=200, depth=5)
  spmm16 = _make_sc_scatter(16, gather=True, ch=200, depth=10, ahead=5)
  spmm128 = _make_sc_scatter(H, gather=True, ch=160, depth=2, ahead=2)

  degp = deg_k(pk200, xp)

  pre = pl.pallas_call(
      _tc_pre_body,
      grid=(_NBLK,),
      in_specs=_row_blocked(((NC, _R, 16), lambda i: (0, i, 0)),
                            ((_R, 16), lambda i: (i, 0))),
      out_specs=_row_blocked(((_R, 1), lambda i: (i, 0)),
                             ((_R, 16), lambda i: (i, 0))),
      out_shape=[jax.ShapeDtypeStruct((N, 1), jnp.float32),
                 jax.ShapeDtypeStruct((N, 16), jnp.float32)],
  )
  dinv, p1 = pre(degp, xp)

  s1 = spmm16(pk200, p1)

  l1 = pl.pallas_call(
      _tc_layer1_body,
      grid=(_NBLK,),
      in_specs=_row_blocked(((NC, _R, 16), lambda i: (0, i, 0)),
                            ((_R, 16), lambda i: (i, 0)),
                            ((_R, 1), lambda i: (i, 0)),
                            ((16, H), lambda i: (0, 0)),
                            ((1, H), lambda i: (0, 0)),
                            ((H, H), lambda i: (0, 0))),
      out_specs=pl.BlockSpec((_R, H), lambda i: (i, 0)),
      out_shape=jax.ShapeDtypeStruct((N, H), jnp.float32),
  )
  p2 = l1(s1, p1, dinv, W1p, b1.reshape(1, H), W2)

  layer = pl.pallas_call(
      _tc_layer_body,
      grid=(_NBLK,),
      in_specs=_row_blocked(((NC, _R, H), lambda i: (0, i, 0)),
                            ((_R, H), lambda i: (i, 0)),
                            ((_R, 1), lambda i: (i, 0)),
                            ((1, H), lambda i: (0, 0)),
                            ((H, H), lambda i: (0, 0))),
      out_specs=pl.BlockSpec((_R, H), lambda i: (i, 0)),
      out_shape=jax.ShapeDtypeStruct((N, H), jnp.float32),
  )

  s2 = spmm128(pk160, p2)
  p3 = layer(s2, p2, dinv, b2.reshape(1, H), W3)
  s3 = spmm128(pk160, p3)
  p4 = layer(s3, p3, dinv, b3.reshape(1, H), W4)
  s4 = spmm128(pk160, p4)

  fin = pl.pallas_call(
      _tc_final_body,
      grid=(_NBLK,),
      in_specs=_row_blocked(((NC, _R, H), lambda i: (0, i, 0)),
                            ((_R, H), lambda i: (i, 0)),
                            ((_R, 1), lambda i: (i, 0)),
                            ((1, H), lambda i: (0, 0)),
                            ((_R, 1), lambda i: (i, 0)),
                            ((H, 8), lambda i: (0, 0)),
                            ((1, 8), lambda i: (0, 0))),
      out_specs=pl.BlockSpec((G, 8), lambda i: (0, 0)),
      out_shape=jax.ShapeDtypeStruct((G, 8), jnp.float32),
      scratch_shapes=[pltpu.VMEM((G, H), jnp.float32),
                      pltpu.VMEM((G, H), jnp.float32)],
  )
  out8 = fin(s4, p4, dinv, b4.reshape(1, H), batch2, Wlp, blp)
  return out8[:, :NCLS]


# speedup vs baseline: 1.9093x; 1.0044x over previous
import functools

import jax
import jax.numpy as jnp
from jax import lax
from jax.experimental import pallas as pl
from jax.experimental.pallas import tpu as pltpu
from jax.experimental.pallas import tpu_sc as plsc

N = 10000
E = 640000
G = 64
H = 128
NCLS = 5

NC = 2
NS = 16
EPC = E // NC
EPT = EPC // NS
ROWS_PT = 640
LAST_ROWS = N - 15 * ROWS_PT


def _make_sc_scatter(width, gather, ch, depth, ahead=0):
  ZCH = 80
  n_full = ROWS_PT // ZCH
  n_last = LAST_ROWS // ZCH
  nchunk = EPT // ch
  rem = nchunk % depth
  ngrp = (nchunk - rem) // depth

  def body(eidx_hbm, tbl_hbm, out_hbm, idx2, rows, acc, *sems):
    gsem = sems[:depth]
    ssem = sems[depth:]
    c = lax.axis_index("c")
    s = lax.axis_index("s")
    cbase = c * (EPC // ch) + s * nchunk
    row0 = s * ROWS_PT

    fill = jnp.zeros((16,), jnp.float32)

    @pl.loop(0, ZCH)
    def _zero_rows(i):
      for k in range(width // 16):
        rows[0, i, pl.ds(k * 16, 16)] = fill

    for r in range(n_full):
      @pl.when((s < NS - 1) | (r < n_last))
      def _zcp():
        pltpu.sync_copy(rows.at[0, pl.ds(0, ZCH)],
                        acc.at[pl.ds(row0 + r * ZCH, ZCH)])

    if not gather:
      one = jnp.ones((16,), jnp.float32)

      @pl.loop(0, ch)
      def _fill_ones(i):
        for k in range(width // 16):
          rows[0, i, pl.ds(k * 16, 16)] = one

    plsc.subcore_barrier()

    if gather:
      def load_and_fire(chunk, k):
        pltpu.sync_copy(eidx_hbm.at[cbase + chunk], idx2.at[k])
        pltpu.async_copy(tbl_hbm.at[idx2.at[k, 0]], rows.at[k], gsem[k])

      for k in range(rem):
        load_and_fire(k, 0)
        pltpu.make_async_copy(tbl_hbm.at[idx2.at[0, 0]], rows.at[0],
                              gsem[0]).wait()
        pltpu.sync_copy(rows.at[0], acc.at[idx2.at[0, 1]], add=True)

      for k in range(ahead):
        load_and_fire(rem + k, k)

      nmain = nchunk - rem

      @pl.loop(0, ngrp)
      def _grp(g):
        for b in range(depth):
          cc = g * depth + b
          pltpu.make_async_copy(tbl_hbm.at[idx2.at[b, 0]], rows.at[b],
                                gsem[b]).wait()
          pltpu.async_copy(rows.at[b], acc.at[idx2.at[b, 1]], ssem[b],
                           add=True)
          kp = (b + ahead) % depth
          cp = cc + ahead

          @pl.when((cp < nmain) & (cp >= depth))
          def _reuse():
            pltpu.make_async_copy(rows.at[kp], acc.at[idx2.at[kp, 1]],
                                  ssem[kp]).wait()

          @pl.when(cp < nmain)
          def _prep():
            load_and_fire(rem + cp, kp)

      for k in range(depth):
        pltpu.make_async_copy(rows.at[k], acc.at[idx2.at[k, 1]],
                              ssem[k]).wait()
    else:
      def fire_idx(chunk, b):
        pltpu.async_copy(eidx_hbm.at[cbase + chunk], idx2.at[b], sems[b])

      def wait_idx(chunk, b):
        pltpu.make_async_copy(eidx_hbm.at[cbase + chunk], idx2.at[b],
                              sems[b]).wait()

      for b in range(depth):
        fire_idx(b, b)

      @pl.loop(0, ngrp)
      def _grp(g):
        for b in range(depth):
          chunk = g * depth + b
          wait_idx(chunk, b)
          pltpu.sync_copy(rows.at[0], acc.at[idx2.at[b, 1]], add=True)

          @pl.when(g + 1 < ngrp)
          def _prefetch():
            fire_idx(chunk + depth, b)

    plsc.subcore_barrier()

    @pl.when(s < NS - 1)
    def _wr_full():
      pltpu.sync_copy(acc.at[pl.ds(row0, ROWS_PT)],
                      out_hbm.at[c, pl.ds(row0, ROWS_PT)])

    @pl.when(s == NS - 1)
    def _wr_last():
      pltpu.sync_copy(acc.at[pl.ds(row0, LAST_ROWS)],
                      out_hbm.at[c, pl.ds(row0, LAST_ROWS)])

  mesh = plsc.VectorSubcoreMesh(core_axis_name="c", subcore_axis_name="s")
  return pl.kernel(
      body,
      compiler_params=pltpu.CompilerParams(use_tc_tiling_on_sc=False),
      out_type=jax.ShapeDtypeStruct((NC, N, width), jnp.float32),
      mesh=mesh,
      scratch_types=[
          pltpu.VMEM((depth, 2, ch), jnp.int32),
          pltpu.VMEM((depth, ch, width), jnp.float32),
          pltpu.VMEM_SHARED((N, width), jnp.float32),
      ] + [pltpu.SemaphoreType.DMA] * (2 * depth if gather else depth),
  )


_R = 10000
_NBLK = N // _R


def _tc_pre_body(degp_ref, xp_ref, dinv_ref, p1_ref):
  deg16 = degp_ref[0] + degp_ref[1] + 1.0
  dinv16 = lax.rsqrt(deg16)
  p1_ref[...] = dinv16 * xp_ref[...]
  dinv_ref[...] = dinv16[:, 0:1]


def _tc_layer1_body(sp_ref, p1_ref, dinv_ref, w1_ref, b1_ref, w2_ref, p2_ref):
  dinv = dinv_ref[...]
  z = dinv * (sp_ref[0] + sp_ref[1] + p1_ref[...])
  h = jnp.maximum(jnp.dot(z, w1_ref[...], preferred_element_type=jnp.float32)
                  + b1_ref[...], 0.0)
  p2_ref[...] = dinv * jnp.dot(h, w2_ref[...], preferred_element_type=jnp.float32)


def _tc_layer_body(sp_ref, p_ref, dinv_ref, b_ref, wn_ref, pn_ref):
  dinv = dinv_ref[...]
  h = jnp.maximum(dinv * (sp_ref[0] + sp_ref[1] + p_ref[...]) + b_ref[...], 0.0)
  pn_ref[...] = dinv * jnp.dot(h, wn_ref[...], preferred_element_type=jnp.float32)


def _tc_final_body(sp_ref, p_ref, dinv_ref, b_ref, batch_ref, wl_ref, bl_ref,
                   out_ref, sums, cnt):
  i = pl.program_id(0)

  @pl.when(i == 0)
  def _init():
    sums[...] = jnp.zeros_like(sums)
    cnt[...] = jnp.zeros_like(cnt)

  dinv = dinv_ref[...]
  h = jnp.maximum(dinv * (sp_ref[0] + sp_ref[1] + p_ref[...]) + b_ref[...], 0.0)
  iota = lax.broadcasted_iota(jnp.int32, (1, G), 1)
  oh = (batch_ref[...] == iota).astype(jnp.float32)
  dn = (((0,), (0,)), ((), ()))
  sums[...] += lax.dot_general(oh, h, dn, preferred_element_type=jnp.float32)
  cnt[...] += lax.dot_general(oh, jnp.ones((_R, H), jnp.float32), dn,
                              preferred_element_type=jnp.float32)

  @pl.when(i == _NBLK - 1)
  def _fin():
    pooled = sums[...] / jnp.maximum(cnt[...], 1.0)
    logits = jnp.dot(pooled, wl_ref[...], preferred_element_type=jnp.float32) + bl_ref[...]
    out_ref[...] = jax.nn.sigmoid(logits)


def _row_blocked(*shapes_maps):
  return [pl.BlockSpec(s, m) for s, m in shapes_maps]


def kernel(x, edge_index, batch, W1, b1, W2, b2, W3, b3, W4, b4, Wlin, blin):
  src = edge_index[0]
  dst = edge_index[1]
  pk160 = jnp.stack([src.reshape(-1, 160), dst.reshape(-1, 160)], axis=1)
  pk400 = jnp.stack([src.reshape(-1, 400), dst.reshape(-1, 400)], axis=1)

  xp = jnp.zeros((N, 16), jnp.float32).at[:, :4].set(x)
  W1p = jnp.zeros((16, H), jnp.float32).at[:4, :].set(W1)
  Wlp = jnp.zeros((H, 8), jnp.float32).at[:, :NCLS].set(Wlin)
  blp = jnp.zeros((1, 8), jnp.float32).at[0, :NCLS].set(blin)
  batch2 = batch.reshape(N, 1)

  deg_k = _make_sc_scatter(16, gather=False, ch=400, depth=5)
  spmm16 = _make_sc_scatter(16, gather=True, ch=400, depth=10, ahead=5)
  spmm128 = _make_sc_scatter(H, gather=True, ch=160, depth=2, ahead=2)

  degp = deg_k(pk400, xp)

  pre = pl.pallas_call(
      _tc_pre_body,
      grid=(_NBLK,),
      in_specs=_row_blocked(((NC, _R, 16), lambda i: (0, i, 0)),
                            ((_R, 16), lambda i: (i, 0))),
      out_specs=_row_blocked(((_R, 1), lambda i: (i, 0)),
                             ((_R, 16), lambda i: (i, 0))),
      out_shape=[jax.ShapeDtypeStruct((N, 1), jnp.float32),
                 jax.ShapeDtypeStruct((N, 16), jnp.float32)],
  )
  dinv, p1 = pre(degp, xp)

  s1 = spmm16(pk400, p1)

  l1 = pl.pallas_call(
      _tc_layer1_body,
      grid=(_NBLK,),
      in_specs=_row_blocked(((NC, _R, 16), lambda i: (0, i, 0)),
                            ((_R, 16), lambda i: (i, 0)),
                            ((_R, 1), lambda i: (i, 0)),
                            ((16, H), lambda i: (0, 0)),
                            ((1, H), lambda i: (0, 0)),
                            ((H, H), lambda i: (0, 0))),
      out_specs=pl.BlockSpec((_R, H), lambda i: (i, 0)),
      out_shape=jax.ShapeDtypeStruct((N, H), jnp.float32),
  )
  p2 = l1(s1, p1, dinv, W1p, b1.reshape(1, H), W2)

  layer = pl.pallas_call(
      _tc_layer_body,
      grid=(_NBLK,),
      in_specs=_row_blocked(((NC, _R, H), lambda i: (0, i, 0)),
                            ((_R, H), lambda i: (i, 0)),
                            ((_R, 1), lambda i: (i, 0)),
                            ((1, H), lambda i: (0, 0)),
                            ((H, H), lambda i: (0, 0))),
      out_specs=pl.BlockSpec((_R, H), lambda i: (i, 0)),
      out_shape=jax.ShapeDtypeStruct((N, H), jnp.float32),
  )

  s2 = spmm128(pk160, p2)
  p3 = layer(s2, p2, dinv, b2.reshape(1, H), W3)
  s3 = spmm128(pk160, p3)
  p4 = layer(s3, p3, dinv, b3.reshape(1, H), W4)
  s4 = spmm128(pk160, p4)

  fin = pl.pallas_call(
      _tc_final_body,
      grid=(_NBLK,),
      in_specs=_row_blocked(((NC, _R, H), lambda i: (0, i, 0)),
                            ((_R, H), lambda i: (i, 0)),
                            ((_R, 1), lambda i: (i, 0)),
                            ((1, H), lambda i: (0, 0)),
                            ((_R, 1), lambda i: (i, 0)),
                            ((H, 8), lambda i: (0, 0)),
                            ((1, 8), lambda i: (0, 0))),
      out_specs=pl.BlockSpec((G, 8), lambda i: (0, 0)),
      out_shape=jax.ShapeDtypeStruct((G, 8), jnp.float32),
      scratch_shapes=[pltpu.VMEM((G, H), jnp.float32),
                      pltpu.VMEM((G, H), jnp.float32)],
  )
  out8 = fin(s4, p4, dinv, b4.reshape(1, H), batch2, Wlp, blp)
  return out8[:, :NCLS]


# speedup vs baseline: 2.0275x; 1.0619x over previous
import functools

import jax
import jax.numpy as jnp
from jax import lax
from jax.experimental import pallas as pl
from jax.experimental.pallas import tpu as pltpu
from jax.experimental.pallas import tpu_sc as plsc

N = 10000
E = 640000
G = 64
H = 128
NCLS = 5

NC = 2
NS = 16
EPC = E // NC
EPT = EPC // NS
ROWS_PT = 640
LAST_ROWS = N - 15 * ROWS_PT


def _make_sc_scatter(width, gather, ch, depth, ahead=0, sep_idx=False):
  ZCH = 80
  n_full = ROWS_PT // ZCH
  n_last = LAST_ROWS // ZCH
  nchunk = EPT // ch
  ID = 2 * depth
  rem = nchunk % (ID if sep_idx else depth)
  ngrp = (nchunk - rem) // depth

  def body(eidx_hbm, tbl_hbm, out_hbm, idx2, rows, acc, *sems):
    gsem = sems[:depth]
    ssem = sems[depth:]
    c = lax.axis_index("c")
    s = lax.axis_index("s")
    cbase = c * (EPC // ch) + s * nchunk
    row0 = s * ROWS_PT

    fill = jnp.zeros((16,), jnp.float32)

    @pl.loop(0, ZCH)
    def _zero_rows(i):
      for k in range(width // 16):
        rows[0, i, pl.ds(k * 16, 16)] = fill

    for r in range(n_full):
      @pl.when((s < NS - 1) | (r < n_last))
      def _zcp():
        pltpu.sync_copy(rows.at[0, pl.ds(0, ZCH)],
                        acc.at[pl.ds(row0 + r * ZCH, ZCH)])

    if not gather:
      one = jnp.ones((16,), jnp.float32)

      @pl.loop(0, ch)
      def _fill_ones(i):
        for k in range(width // 16):
          rows[0, i, pl.ds(k * 16, 16)] = one

    plsc.subcore_barrier()

    if gather and sep_idx:
      isem = sems[2 * depth:]

      for k in range(rem):
        pltpu.sync_copy(eidx_hbm.at[cbase + k], idx2.at[0])
        pltpu.async_copy(tbl_hbm.at[idx2.at[0, 0]], rows.at[0], gsem[0]).wait()
        pltpu.sync_copy(rows.at[0], acc.at[idx2.at[0, 1]], add=True)

      nmain = nchunk - rem

      for u in range(depth):
        pltpu.sync_copy(eidx_hbm.at[cbase + rem + u], idx2.at[u])
        pltpu.async_copy(tbl_hbm.at[idx2.at[u, 0]], rows.at[u], gsem[u])
      for u in range(depth, ID):
        pltpu.async_copy(eidx_hbm.at[cbase + rem + u], idx2.at[u], isem[u])

      @pl.loop(0, nmain // ID)
      def _grp(q):
        for u in range(ID):
          b = u % depth
          cc = q * ID + u
          pltpu.make_async_copy(tbl_hbm.at[idx2.at[u, 0]], rows.at[b],
                                gsem[b]).wait()
          pltpu.async_copy(rows.at[b], acc.at[idx2.at[u, 1]], ssem[b],
                           add=True)
          pltpu.make_async_copy(rows.at[b], acc.at[idx2.at[u, 1]],
                                ssem[b]).wait()
          cp = cc + depth
          up = (u + depth) % ID

          @pl.when(cp < nmain)
          def _next_gather():
            pltpu.make_async_copy(eidx_hbm.at[cbase + rem + cp],
                                  idx2.at[up], isem[up]).wait()
            pltpu.async_copy(tbl_hbm.at[idx2.at[up, 0]], rows.at[b], gsem[b])

          cf = cc + ID

          @pl.when(cf < nmain)
          def _next_idx():
            pltpu.async_copy(eidx_hbm.at[cbase + rem + cf], idx2.at[u],
                             isem[u])

    elif gather:
      def load_and_fire(chunk, k):
        pltpu.sync_copy(eidx_hbm.at[cbase + chunk], idx2.at[k])
        pltpu.async_copy(tbl_hbm.at[idx2.at[k, 0]], rows.at[k], gsem[k])

      for k in range(rem):
        load_and_fire(k, 0)
        pltpu.make_async_copy(tbl_hbm.at[idx2.at[0, 0]], rows.at[0],
                              gsem[0]).wait()
        pltpu.sync_copy(rows.at[0], acc.at[idx2.at[0, 1]], add=True)

      for k in range(ahead):
        load_and_fire(rem + k, k)

      nmain = nchunk - rem

      @pl.loop(0, ngrp)
      def _grp(g):
        for b in range(depth):
          cc = g * depth + b
          pltpu.make_async_copy(tbl_hbm.at[idx2.at[b, 0]], rows.at[b],
                                gsem[b]).wait()
          pltpu.async_copy(rows.at[b], acc.at[idx2.at[b, 1]], ssem[b],
                           add=True)
          kp = (b + ahead) % depth
          cp = cc + ahead

          @pl.when((cp < nmain) & (cp >= depth))
          def _reuse():
            pltpu.make_async_copy(rows.at[kp], acc.at[idx2.at[kp, 1]],
                                  ssem[kp]).wait()

          @pl.when(cp < nmain)
          def _prep():
            load_and_fire(rem + cp, kp)

      for k in range(depth):
        pltpu.make_async_copy(rows.at[k], acc.at[idx2.at[k, 1]],
                              ssem[k]).wait()
    else:
      def fire_idx(chunk, b):
        pltpu.async_copy(eidx_hbm.at[cbase + chunk], idx2.at[b], sems[b])

      def wait_idx(chunk, b):
        pltpu.make_async_copy(eidx_hbm.at[cbase + chunk], idx2.at[b],
                              sems[b]).wait()

      for b in range(depth):
        fire_idx(b, b)

      @pl.loop(0, ngrp)
      def _grp(g):
        for b in range(depth):
          chunk = g * depth + b
          wait_idx(chunk, b)
          pltpu.sync_copy(rows.at[0], acc.at[idx2.at[b, 1]], add=True)

          @pl.when(g + 1 < ngrp)
          def _prefetch():
            fire_idx(chunk + depth, b)

    plsc.subcore_barrier()

    @pl.when(s < NS - 1)
    def _wr_full():
      pltpu.sync_copy(acc.at[pl.ds(row0, ROWS_PT)],
                      out_hbm.at[c, pl.ds(row0, ROWS_PT)])

    @pl.when(s == NS - 1)
    def _wr_last():
      pltpu.sync_copy(acc.at[pl.ds(row0, LAST_ROWS)],
                      out_hbm.at[c, pl.ds(row0, LAST_ROWS)])

  mesh = plsc.VectorSubcoreMesh(core_axis_name="c", subcore_axis_name="s")
  return pl.kernel(
      body,
      compiler_params=pltpu.CompilerParams(use_tc_tiling_on_sc=False),
      out_type=jax.ShapeDtypeStruct((NC, N, width), jnp.float32),
      mesh=mesh,
      scratch_types=[
          pltpu.VMEM((ID if sep_idx else depth, 2, ch), jnp.int32),
          pltpu.VMEM((depth, ch, width), jnp.float32),
          pltpu.VMEM_SHARED((N, width), jnp.float32),
      ] + [pltpu.SemaphoreType.DMA] * (
          2 * depth + ID if (gather and sep_idx)
          else (2 * depth if gather else depth)),
  )


_R = 10000
_NBLK = N // _R


def _tc_pre_body(degp_ref, xp_ref, dinv_ref, p1_ref):
  deg16 = degp_ref[0] + degp_ref[1] + 1.0
  dinv16 = lax.rsqrt(deg16)
  p1_ref[...] = dinv16 * xp_ref[...]
  dinv_ref[...] = dinv16[:, 0:1]


def _tc_layer1_body(sp_ref, p1_ref, dinv_ref, w1_ref, b1_ref, w2_ref, p2_ref):
  dinv = dinv_ref[...]
  z = dinv * (sp_ref[0] + sp_ref[1] + p1_ref[...])
  h = jnp.maximum(jnp.dot(z, w1_ref[...], preferred_element_type=jnp.float32)
                  + b1_ref[...], 0.0)
  p2_ref[...] = dinv * jnp.dot(h, w2_ref[...], preferred_element_type=jnp.float32)


def _tc_layer_body(sp_ref, p_ref, dinv_ref, b_ref, wn_ref, pn_ref):
  dinv = dinv_ref[...]
  h = jnp.maximum(dinv * (sp_ref[0] + sp_ref[1] + p_ref[...]) + b_ref[...], 0.0)
  pn_ref[...] = dinv * jnp.dot(h, wn_ref[...], preferred_element_type=jnp.float32)


def _tc_final_body(sp_ref, p_ref, dinv_ref, b_ref, batch_ref, wl_ref, bl_ref,
                   out_ref, sums, cnt):
  i = pl.program_id(0)

  @pl.when(i == 0)
  def _init():
    sums[...] = jnp.zeros_like(sums)
    cnt[...] = jnp.zeros_like(cnt)

  dinv = dinv_ref[...]
  h = jnp.maximum(dinv * (sp_ref[0] + sp_ref[1] + p_ref[...]) + b_ref[...], 0.0)
  iota = lax.broadcasted_iota(jnp.int32, (1, G), 1)
  oh = (batch_ref[...] == iota).astype(jnp.float32)
  dn = (((0,), (0,)), ((), ()))
  sums[...] += lax.dot_general(oh, h, dn, preferred_element_type=jnp.float32)
  cnt[...] += lax.dot_general(oh, jnp.ones((_R, H), jnp.float32), dn,
                              preferred_element_type=jnp.float32)

  @pl.when(i == _NBLK - 1)
  def _fin():
    pooled = sums[...] / jnp.maximum(cnt[...], 1.0)
    logits = jnp.dot(pooled, wl_ref[...], preferred_element_type=jnp.float32) + bl_ref[...]
    out_ref[...] = jax.nn.sigmoid(logits)


def _row_blocked(*shapes_maps):
  return [pl.BlockSpec(s, m) for s, m in shapes_maps]


def kernel(x, edge_index, batch, W1, b1, W2, b2, W3, b3, W4, b4, Wlin, blin):
  src = edge_index[0]
  dst = edge_index[1]
  pk160 = jnp.stack([src.reshape(-1, 160), dst.reshape(-1, 160)], axis=1)
  pk400 = jnp.stack([src.reshape(-1, 400), dst.reshape(-1, 400)], axis=1)

  xp = jnp.zeros((N, 16), jnp.float32).at[:, :4].set(x)
  W1p = jnp.zeros((16, H), jnp.float32).at[:4, :].set(W1)
  Wlp = jnp.zeros((H, 8), jnp.float32).at[:, :NCLS].set(Wlin)
  blp = jnp.zeros((1, 8), jnp.float32).at[0, :NCLS].set(blin)
  batch2 = batch.reshape(N, 1)

  deg_k = _make_sc_scatter(16, gather=False, ch=400, depth=5)
  spmm16 = _make_sc_scatter(16, gather=True, ch=400, depth=10, ahead=5)
  spmm128 = _make_sc_scatter(H, gather=True, ch=160, depth=2, ahead=2,
                             sep_idx=True)

  degp = deg_k(pk400, xp)

  pre = pl.pallas_call(
      _tc_pre_body,
      grid=(_NBLK,),
      in_specs=_row_blocked(((NC, _R, 16), lambda i: (0, i, 0)),
                            ((_R, 16), lambda i: (i, 0))),
      out_specs=_row_blocked(((_R, 1), lambda i: (i, 0)),
                             ((_R, 16), lambda i: (i, 0))),
      out_shape=[jax.ShapeDtypeStruct((N, 1), jnp.float32),
                 jax.ShapeDtypeStruct((N, 16), jnp.float32)],
  )
  dinv, p1 = pre(degp, xp)

  s1 = spmm16(pk400, p1)

  l1 = pl.pallas_call(
      _tc_layer1_body,
      grid=(_NBLK,),
      in_specs=_row_blocked(((NC, _R, 16), lambda i: (0, i, 0)),
                            ((_R, 16), lambda i: (i, 0)),
                            ((_R, 1), lambda i: (i, 0)),
                            ((16, H), lambda i: (0, 0)),
                            ((1, H), lambda i: (0, 0)),
                            ((H, H), lambda i: (0, 0))),
      out_specs=pl.BlockSpec((_R, H), lambda i: (i, 0)),
      out_shape=jax.ShapeDtypeStruct((N, H), jnp.float32),
  )
  p2 = l1(s1, p1, dinv, W1p, b1.reshape(1, H), W2)

  layer = pl.pallas_call(
      _tc_layer_body,
      grid=(_NBLK,),
      in_specs=_row_blocked(((NC, _R, H), lambda i: (0, i, 0)),
                            ((_R, H), lambda i: (i, 0)),
                            ((_R, 1), lambda i: (i, 0)),
                            ((1, H), lambda i: (0, 0)),
                            ((H, H), lambda i: (0, 0))),
      out_specs=pl.BlockSpec((_R, H), lambda i: (i, 0)),
      out_shape=jax.ShapeDtypeStruct((N, H), jnp.float32),
  )

  s2 = spmm128(pk160, p2)
  p3 = layer(s2, p2, dinv, b2.reshape(1, H), W3)
  s3 = spmm128(pk160, p3)
  p4 = layer(s3, p3, dinv, b3.reshape(1, H), W4)
  s4 = spmm128(pk160, p4)

  fin = pl.pallas_call(
      _tc_final_body,
      grid=(_NBLK,),
      in_specs=_row_blocked(((NC, _R, H), lambda i: (0, i, 0)),
                            ((_R, H), lambda i: (i, 0)),
                            ((_R, 1), lambda i: (i, 0)),
                            ((1, H), lambda i: (0, 0)),
                            ((_R, 1), lambda i: (i, 0)),
                            ((H, 8), lambda i: (0, 0)),
                            ((1, 8), lambda i: (0, 0))),
      out_specs=pl.BlockSpec((G, 8), lambda i: (0, 0)),
      out_shape=jax.ShapeDtypeStruct((G, 8), jnp.float32),
      scratch_shapes=[pltpu.VMEM((G, H), jnp.float32),
                      pltpu.VMEM((G, H), jnp.float32)],
  )
  out8 = fin(s4, p4, dinv, b4.reshape(1, H), batch2, Wlp, blp)
  return out8[:, :NCLS]
